# Initial kernel scaffold; baseline (speedup 1.0000x reference)
import jax
import jax.numpy as jnp
from jax.experimental import pallas as pl


def kernel(X, edge_index, he_nodes, he_edges, W, b):
    raise NotImplementedError("write your pallas kernel here")


# speedup vs baseline: 3.8619x; 3.8619x over previous
import functools

import jax
import jax.numpy as jnp
from jax import lax
from jax.experimental import pallas as pl
from jax.experimental.pallas import tpu as pltpu
from jax.experimental.pallas import tpu_sc as plsc

N = 10000
E = 320000
D = 128
NE = 2000
EH = 40000

NC = 2
NS = 16
NW = NC * NS
CH = 128
WR = 144

R_G = 10240
R_E = 2048
K_G = -(-E // (NW * CH))
K_E = -(-EH // (NW * CH))


def _sc_scatter_add(table, gidx, sidx, *, R, K):
    rows_per_tile = R // NS
    nzb = rows_per_tile // CH
    mesh = plsc.VectorSubcoreMesh(core_axis_name="c", subcore_axis_name="s")

    @functools.partial(
        pl.kernel,
        mesh=mesh,
        compiler_params=pltpu.CompilerParams(use_tc_tiling_on_sc=False),
        out_type=jax.ShapeDtypeStruct((NC, R, WR), jnp.float32),
        scratch_types=[
            pltpu.VMEM((CH,), jnp.int32),
            pltpu.VMEM((CH,), jnp.int32),
            pltpu.VMEM((CH, WR), jnp.float32),
            pltpu.VMEM((CH, WR), jnp.float32),
            pltpu.VMEM_SHARED((R, WR), jnp.float32),
            pltpu.SemaphoreType.DMA,
        ],
    )
    def k(table_h, gidx_h, sidx_h, out_h, gi_v, si_v, rows_v, z_v, acc_s, sem):
        cid = lax.axis_index("c")
        sid = lax.axis_index("s")
        wid = sid * NC + cid
        base = sid * rows_per_tile

        def zrow(i, c):
            def zcol(j, c2):
                z_v[i, pl.ds(j * 16, 16)] = jnp.zeros((16,), jnp.float32)
                return c2
            return lax.fori_loop(0, WR // 16, zcol, c)
        lax.fori_loop(0, CH, zrow, 0)

        def zstripe(r, c):
            pltpu.sync_copy(z_v, acc_s.at[pl.ds(base + r * CH, CH)])
            return c
        lax.fori_loop(0, nzb, zstripe, 0)
        plsc.subcore_barrier()

        def body(t, c):
            pltpu.sync_copy(gidx_h.at[wid, t], gi_v)
            pltpu.sync_copy(sidx_h.at[wid, t], si_v)
            pltpu.async_copy(table_h.at[gi_v], rows_v, sem).wait()
            pltpu.sync_copy(rows_v, acc_s.at[si_v], add=True)
            return c
        lax.fori_loop(0, K, body, 0)
        plsc.subcore_barrier()

        def wb(r, c):
            pltpu.sync_copy(acc_s.at[pl.ds(base + r * CH, CH)],
                            out_h.at[cid, pl.ds(base + r * CH, CH)])
            return c
        lax.fori_loop(0, nzb, wb, 0)

    return k(table, gidx, sidx)


def _tc_matmul(Xp, Wz, b144):
    BR = 1024

    def mmk(x_ref, w_ref, b_ref, o_ref):
        o_ref[...] = lax.dot(
            x_ref[...], w_ref[...],
            precision=lax.Precision.HIGHEST,
            preferred_element_type=jnp.float32,
        ) + b_ref[0:1, :]

    return pl.pallas_call(
        mmk,
        grid=(R_G // BR,),
        in_specs=[
            pl.BlockSpec((BR, D), lambda i: (i, 0)),
            pl.BlockSpec((D, WR), lambda i: (0, 0)),
            pl.BlockSpec((8, WR), lambda i: (0, 0)),
        ],
        out_specs=pl.BlockSpec((BR, WR), lambda i: (i, 0)),
        out_shape=jax.ShapeDtypeStruct((R_G, WR), jnp.float32),
    )(Xp, Wz, b144)


def _tc_finalize_y(pe):
    def fk(p_ref, o_ref):
        s = p_ref[0] + p_ref[1]
        o_ref[...] = s / jnp.maximum(s[:, 128:129], 1.0)

    return pl.pallas_call(
        fk,
        out_shape=jax.ShapeDtypeStruct((R_E, WR), jnp.float32),
    )(pe)


def _tc_combine(pg, ph):
    BR = 1024

    def ck(pg_ref, ph_ref, o_ref):
        sg = pg_ref[0] + pg_ref[1]
        sh = ph_ref[0] + ph_ref[1]
        xg = sg[:, :D] / jnp.maximum(sg[:, 128:129], 1.0)
        xh = sh[:, :D] / jnp.maximum(sh[:, 128:129], 1.0)
        o_ref[...] = jnp.maximum(xg * 0.1 + xh * 0.9, 0.0)

    return pl.pallas_call(
        ck,
        grid=(R_G // BR,),
        in_specs=[
            pl.BlockSpec((2, BR, WR), lambda i: (0, i, 0)),
            pl.BlockSpec((2, BR, WR), lambda i: (0, i, 0)),
        ],
        out_specs=pl.BlockSpec((BR, D), lambda i: (i, 0)),
        out_shape=jax.ShapeDtypeStruct((R_G, D), jnp.float32),
    )(pg, ph)


def _pad_pairs(g, s, K, gpad, spad):
    tot = NW * K * CH
    g = jnp.concatenate(
        [g.astype(jnp.int32), jnp.full((tot - g.shape[0],), gpad, jnp.int32)])
    s = jnp.concatenate(
        [s.astype(jnp.int32), jnp.full((tot - s.shape[0],), spad, jnp.int32)])
    return g.reshape(NW, K, CH), s.reshape(NW, K, CH)


def kernel(X, edge_index, he_nodes, he_edges, W, b):
    Xp = jnp.pad(X, ((0, R_G - N), (0, 0)))
    Wz = jnp.pad(W.T, ((0, 0), (0, WR - D)))
    b144 = jnp.zeros((8, WR), jnp.float32).at[0, :D].set(b).at[0, D].set(1.0)

    H = _tc_matmul(Xp, Wz, b144)

    gv, sv = _pad_pairs(edge_index[0], edge_index[1], K_G, N, R_G - 1)
    pg = _sc_scatter_add(H, gv, sv, R=R_G, K=K_G)

    ge, se = _pad_pairs(he_nodes, he_edges, K_E, N, R_E - 1)
    pe = _sc_scatter_add(H, ge, se, R=R_E, K=K_E)
    Yp = _tc_finalize_y(pe)

    gh, sh = _pad_pairs(he_edges, he_nodes, K_E, R_E - 1, R_G - 1)
    ph = _sc_scatter_add(Yp, gh, sh, R=R_G, K=K_E)

    out = _tc_combine(pg, ph)
    return out[:N]
